# baseline (speedup 1.0000x reference)
import functools

import jax
import jax.numpy as jnp
from jax import lax
from jax.experimental import pallas as pl
from jax.experimental.pallas import tpu as pltpu
from jax.experimental.pallas import tpu_sc as plsc

CH = 32
V = CH // 16


def _roi_align_sc(table, props_t, B, N, H, W, C, PH, PW):
    S = B * N * PH * PW
    n_chunks = S // CH
    NW = 32
    base_chunks = n_chunks // NW
    extra = n_chunks % NW
    NB = B * N
    PAIRS = (base_chunks + 2) // 2
    HC = C // 2

    mesh = plsc.VectorSubcoreMesh(core_axis_name="c", subcore_axis_name="s")

    @functools.partial(
        pl.kernel,
        mesh=mesh,
        compiler_params=pltpu.CompilerParams(needs_layout_passes=False),
        out_type=jax.ShapeDtypeStruct((2 * S, HC), jnp.float32),
        scratch_types=[
            pltpu.VMEM((4 * NB,), jnp.float32),
            pltpu.VMEM((8 * CH,), jnp.int32),
            pltpu.VMEM((8 * CH,), jnp.int32),
            pltpu.VMEM((4 * CH,), jnp.float32),
            pltpu.VMEM((4 * CH,), jnp.float32),
            pltpu.VMEM((8 * CH, HC), jnp.float32),
            pltpu.VMEM((8 * CH, HC), jnp.float32),
            pltpu.VMEM((2 * CH, HC), jnp.float32),
            pltpu.VMEM((2 * CH, HC), jnp.float32),
            pltpu.SemaphoreType.DMA,
            pltpu.SemaphoreType.DMA,
            pltpu.SemaphoreType.DMA,
            pltpu.SemaphoreType.DMA,
        ],
    )
    def sc_kernel(table_hbm, props_hbm, out_hbm, props_v, idx0, idx1,
                  w0, w1, rows0, rows1, outv0, outv1,
                  sem0, sem1, osem0, osem1):
        idx = [idx0, idx1]
        wv = [w0, w1]
        rows = [rows0, rows1]
        outv = [outv0, outv1]
        sems = [sem0, sem1]
        osems = [osem0, osem1]

        cid = lax.axis_index("c")
        sid = lax.axis_index("s")
        wid = sid * 2 + cid
        pltpu.sync_copy(props_hbm, props_v)
        n = base_chunks + jnp.where(wid < extra, 1, 0)

        yscale = jnp.float32((H - 1) / (PH - 1))
        xscale = jnp.float32((W - 1) / (PW - 1))
        half = jnp.float32(0.5)
        r_pp = jnp.float32(1.0 / (PH * PW))
        r_pw = jnp.float32(1.0 / PW)
        r_n = jnp.float32(1.0 / N)

        def fire(t, b):
            @pl.when(t < n)
            def _():
                chunk = wid + t * NW
                for v in range(V):
                    iota = lax.iota(jnp.int32, 16)
                    s = chunk * CH + v * 16 + iota
                    sf = s.astype(jnp.float32) + half
                    box = (sf * r_pp).astype(jnp.int32)
                    r = s - box * jnp.int32(PH * PW)
                    rf = r.astype(jnp.float32) + half
                    py = (rf * r_pw).astype(jnp.int32)
                    px = r - py * jnp.int32(PW)
                    bb = ((box.astype(jnp.float32) + half)
                          * r_n).astype(jnp.int32)

                    x1 = plsc.load_gather(props_v, [box])
                    y1 = plsc.load_gather(props_v, [box + NB])
                    x2 = plsc.load_gather(props_v, [box + 2 * NB])
                    y2 = plsc.load_gather(props_v, [box + 3 * NB])

                    hs = (y2 - y1) * yscale
                    ws = (x2 - x1) * xscale
                    in_y = (y1 * jnp.float32(H - 1)
                            + py.astype(jnp.float32) * hs)
                    in_x = (x1 * jnp.float32(W - 1)
                            + px.astype(jnp.float32) * ws)
                    in_y = jnp.minimum(jnp.maximum(in_y, jnp.float32(0.0)),
                                       jnp.float32(H - 1))
                    in_x = jnp.minimum(jnp.maximum(in_x, jnp.float32(0.0)),
                                       jnp.float32(W - 1))
                    y_lo = in_y.astype(jnp.int32)
                    x_lo = in_x.astype(jnp.int32)
                    ly = in_y - y_lo.astype(jnp.float32)
                    lx = in_x - x_lo.astype(jnp.float32)
                    y_hi = jnp.minimum(y_lo + 1, jnp.int32(H - 1))
                    x_hi = jnp.minimum(x_lo + 1, jnp.int32(W - 1))

                    row_lo = bb * jnp.int32(H * W) + y_lo * jnp.int32(W)
                    row_hi = bb * jnp.int32(H * W) + y_hi * jnp.int32(W)
                    pos = 2 * (v * 16 + iota)
                    for k, rk in enumerate((row_lo + x_lo, row_lo + x_hi,
                                            row_hi + x_lo, row_hi + x_hi)):
                        e = 2 * rk
                        plsc.store_scatter(idx[b], [pos + 2 * CH * k], e)
                        plsc.store_scatter(idx[b], [pos + (2 * CH * k + 1)],
                                           e + 1)

                    o = v * 16
                    one = jnp.float32(1.0)
                    wv[b][pl.ds(o, 16)] = (one - ly) * (one - lx)
                    wv[b][pl.ds(CH + o, 16)] = (one - ly) * lx
                    wv[b][pl.ds(2 * CH + o, 16)] = ly * (one - lx)
                    wv[b][pl.ds(3 * CH + o, 16)] = ly * lx

                pltpu.async_copy(table_hbm.at[idx[b].at[pl.ds(0, 4 * CH)]],
                                 rows[b].at[pl.ds(0, 4 * CH)], sems[b])
                pltpu.async_copy(table_hbm.at[idx[b].at[pl.ds(4 * CH, 4 * CH)]],
                                 rows[b].at[pl.ds(4 * CH, 4 * CH)], sems[b])

        def consume(t, b):
            @pl.when(t < n)
            def _():
                pltpu.make_async_copy(
                    table_hbm.at[idx[b].at[pl.ds(0, 4 * CH)]],
                    rows[b].at[pl.ds(0, 4 * CH)], sems[b]).wait()
                pltpu.make_async_copy(
                    table_hbm.at[idx[b].at[pl.ds(4 * CH, 4 * CH)]],
                    rows[b].at[pl.ds(4 * CH, 4 * CH)], sems[b]).wait()

                @pl.when(t >= 2)
                def _():
                    s0p = (wid + (t - 2) * NW) * CH * 2
                    pltpu.make_async_copy(
                        outv[b], out_hbm.at[pl.ds(s0p, 2 * CH)],
                        osems[b]).wait()

                zero = jnp.zeros((16,), jnp.int32)
                rb = rows[b]
                wb = wv[b]
                ob = outv[b]

                def sample_body(i, c2):
                    i_splat = zero + i
                    wtl = plsc.load_gather(wb, [i_splat])
                    wtr = plsc.load_gather(wb, [i_splat + CH])
                    wbl = plsc.load_gather(wb, [i_splat + 2 * CH])
                    wbr = plsc.load_gather(wb, [i_splat + 3 * CH])
                    i2 = 2 * i
                    for hf in range(2):
                        for cq in range(HC // 16):
                            sl = pl.ds(cq * 16, 16)
                            ob[i2 + hf, sl] = (
                                wtl * rb[i2 + hf, sl]
                                + wtr * rb[i2 + hf + 2 * CH, sl]
                                + wbl * rb[i2 + hf + 4 * CH, sl]
                                + wbr * rb[i2 + hf + 6 * CH, sl])
                    return c2

                lax.fori_loop(0, CH, sample_body, 0)
                s0 = (wid + t * NW) * CH * 2
                pltpu.async_copy(ob, out_hbm.at[pl.ds(s0, 2 * CH)], osems[b])

        fire(jnp.int32(0), 0)

        def pair_body(p, carry):
            t0 = 2 * p
            fire(t0 + 1, 1)
            consume(t0, 0)
            fire(t0 + 2, 0)
            consume(t0 + 1, 1)
            return carry

        lax.fori_loop(0, PAIRS, pair_body, 0)

        for par in range(2):
            t_par = (n - 1) - jnp.bitwise_and(n - 1 - par, 1)

            @pl.when(t_par >= 0)
            def _(par=par, t_par=t_par):
                s0p = (wid + t_par * NW) * CH * 2
                pltpu.make_async_copy(
                    outv[par], out_hbm.at[pl.ds(s0p, 2 * CH)],
                    osems[par]).wait()

    return sc_kernel(table, props_t)


def kernel(feature, proposals):
    B, H, W, C = feature.shape
    N = proposals.shape[1]
    PH = PW = 7
    props_t = proposals.reshape(B * N, 4).T.reshape(-1)
    table = feature.reshape(B * H * W * 2, C // 2)
    out = _roi_align_sc(table, props_t, B, N, H, W, C, PH, PW)
    return out.reshape(B, N, PH, PW, C)


# speedup vs baseline: 1.0591x; 1.0591x over previous
import functools

import jax
import jax.numpy as jnp
from jax import lax
from jax.experimental import pallas as pl
from jax.experimental.pallas import tpu as pltpu
from jax.experimental.pallas import tpu_sc as plsc

CH = 32
V = CH // 16


def _roi_align_sc(table, props_t, B, N, H, W, C, PH, PW):
    S = B * N * PH * PW
    n_chunks = S // CH
    NW = 32
    base_chunks = n_chunks // NW
    extra = n_chunks % NW
    NB = B * N
    PAIRS = (base_chunks + 2) // 2

    mesh = plsc.VectorSubcoreMesh(core_axis_name="c", subcore_axis_name="s")

    @functools.partial(
        pl.kernel,
        mesh=mesh,
        compiler_params=pltpu.CompilerParams(needs_layout_passes=False),
        out_type=jax.ShapeDtypeStruct((S, C), jnp.float32),
        scratch_types=[
            pltpu.VMEM((4 * NB,), jnp.float32),
            pltpu.VMEM((4 * CH,), jnp.int32),
            pltpu.VMEM((4 * CH,), jnp.int32),
            pltpu.VMEM((4 * CH,), jnp.float32),
            pltpu.VMEM((4 * CH,), jnp.float32),
            pltpu.VMEM((4 * CH, C), jnp.float32),
            pltpu.VMEM((4 * CH, C), jnp.float32),
            pltpu.VMEM((CH, C), jnp.float32),
            pltpu.VMEM((CH, C), jnp.float32),
            pltpu.SemaphoreType.DMA,
            pltpu.SemaphoreType.DMA,
            pltpu.SemaphoreType.DMA,
            pltpu.SemaphoreType.DMA,
        ],
    )
    def sc_kernel(table_hbm, props_hbm, out_hbm, props_v, idx0, idx1,
                  w0, w1, rows0, rows1, outv0, outv1,
                  sem0, sem1, osem0, osem1):
        idx = [idx0, idx1]
        wv = [w0, w1]
        rows = [rows0, rows1]
        outv = [outv0, outv1]
        sems = [sem0, sem1]
        osems = [osem0, osem1]

        cid = lax.axis_index("c")
        sid = lax.axis_index("s")
        wid = sid * 2 + cid
        pltpu.sync_copy(props_hbm, props_v)
        n = base_chunks + jnp.where(wid < extra, 1, 0)

        yscale = jnp.float32((H - 1) / (PH - 1))
        xscale = jnp.float32((W - 1) / (PW - 1))
        half = jnp.float32(0.5)
        r_pp = jnp.float32(1.0 / (PH * PW))
        r_pw = jnp.float32(1.0 / PW)
        r_n = jnp.float32(1.0 / N)

        def fire(t, b):
            @pl.when(t < n)
            def _():
                chunk = wid + t * NW
                for v in range(V):
                    s = chunk * CH + v * 16 + lax.iota(jnp.int32, 16)
                    sf = s.astype(jnp.float32) + half
                    box = (sf * r_pp).astype(jnp.int32)
                    r = s - box * jnp.int32(PH * PW)
                    rf = r.astype(jnp.float32) + half
                    py = (rf * r_pw).astype(jnp.int32)
                    px = r - py * jnp.int32(PW)
                    bb = ((box.astype(jnp.float32) + half)
                          * r_n).astype(jnp.int32)

                    x1 = plsc.load_gather(props_v, [box])
                    y1 = plsc.load_gather(props_v, [box + NB])
                    x2 = plsc.load_gather(props_v, [box + 2 * NB])
                    y2 = plsc.load_gather(props_v, [box + 3 * NB])

                    hs = (y2 - y1) * yscale
                    ws = (x2 - x1) * xscale
                    in_y = (y1 * jnp.float32(H - 1)
                            + py.astype(jnp.float32) * hs)
                    in_x = (x1 * jnp.float32(W - 1)
                            + px.astype(jnp.float32) * ws)
                    in_y = jnp.minimum(jnp.maximum(in_y, jnp.float32(0.0)),
                                       jnp.float32(H - 1))
                    in_x = jnp.minimum(jnp.maximum(in_x, jnp.float32(0.0)),
                                       jnp.float32(W - 1))
                    y_lo = in_y.astype(jnp.int32)
                    x_lo = in_x.astype(jnp.int32)
                    ly = in_y - y_lo.astype(jnp.float32)
                    lx = in_x - x_lo.astype(jnp.float32)
                    y_hi = jnp.minimum(y_lo + 1, jnp.int32(H - 1))
                    x_hi = jnp.minimum(x_lo + 1, jnp.int32(W - 1))

                    row_lo = bb * jnp.int32(H * W) + y_lo * jnp.int32(W)
                    row_hi = bb * jnp.int32(H * W) + y_hi * jnp.int32(W)
                    o = v * 16
                    idx[b][pl.ds(o, 16)] = row_lo + x_lo
                    idx[b][pl.ds(CH + o, 16)] = row_lo + x_hi
                    idx[b][pl.ds(2 * CH + o, 16)] = row_hi + x_lo
                    idx[b][pl.ds(3 * CH + o, 16)] = row_hi + x_hi

                    one = jnp.float32(1.0)
                    wv[b][pl.ds(o, 16)] = (one - ly) * (one - lx)
                    wv[b][pl.ds(CH + o, 16)] = (one - ly) * lx
                    wv[b][pl.ds(2 * CH + o, 16)] = ly * (one - lx)
                    wv[b][pl.ds(3 * CH + o, 16)] = ly * lx

                pltpu.async_copy(table_hbm.at[idx[b]], rows[b], sems[b])

        def consume(t, b):
            @pl.when(t < n)
            def _():
                pltpu.make_async_copy(table_hbm.at[idx[b]], rows[b],
                                      sems[b]).wait()

                @pl.when(t >= 2)
                def _():
                    s0p = (wid + (t - 2) * NW) * CH
                    pltpu.make_async_copy(
                        outv[b], out_hbm.at[pl.ds(s0p, CH)], osems[b]).wait()

                zero = jnp.zeros((16,), jnp.int32)
                rb = rows[b]
                wb = wv[b]
                ob = outv[b]

                def sample_body(i, c2):
                    i_splat = zero + i
                    wtl = plsc.load_gather(wb, [i_splat])
                    wtr = plsc.load_gather(wb, [i_splat + CH])
                    wbl = plsc.load_gather(wb, [i_splat + 2 * CH])
                    wbr = plsc.load_gather(wb, [i_splat + 3 * CH])
                    for cq in range(C // 16):
                        sl = pl.ds(cq * 16, 16)
                        ob[i, sl] = (wtl * rb[i, sl]
                                     + wtr * rb[i + CH, sl]
                                     + wbl * rb[i + 2 * CH, sl]
                                     + wbr * rb[i + 3 * CH, sl])
                    return c2

                lax.fori_loop(0, CH, sample_body, 0)
                s0 = (wid + t * NW) * CH
                pltpu.async_copy(ob, out_hbm.at[pl.ds(s0, CH)], osems[b])

        fire(jnp.int32(0), 0)

        def pair_body(p, carry):
            t0 = 2 * p
            fire(t0 + 1, 1)
            consume(t0, 0)
            fire(t0 + 2, 0)
            consume(t0 + 1, 1)
            return carry

        lax.fori_loop(0, PAIRS, pair_body, 0)

        for par in range(2):
            t_par = (n - 1) - jnp.bitwise_and(n - 1 - par, 1)

            @pl.when(t_par >= 0)
            def _(par=par, t_par=t_par):
                s0p = (wid + t_par * NW) * CH
                pltpu.make_async_copy(
                    outv[par], out_hbm.at[pl.ds(s0p, CH)],
                    osems[par]).wait()

    return sc_kernel(table, props_t)


def kernel(feature, proposals):
    B, H, W, C = feature.shape
    N = proposals.shape[1]
    PH = PW = 7
    props_t = proposals.reshape(B * N, 4).T.reshape(-1)
    table = feature.reshape(B * H * W, C)
    out = _roi_align_sc(table, props_t, B, N, H, W, C, PH, PW)
    return out.reshape(B, N, PH, PW, C)
